# baseline (speedup 1.0000x reference)
import functools

import jax
import jax.numpy as jnp
from jax import lax
from jax.experimental import pallas as pl
from jax.experimental.pallas import tpu as pltpu
from jax.experimental.pallas import tpu_sc as plsc

B = 128
N = 8192
L = 16
NC, NS = 2, 16
NW = NC * NS
RPW = B // NW

LO0 = 0x35000000
HI0 = 0x3F800000
TOTAL_ITERS = 28
M1 = 10


def _sc_body(scores_hbm, seqlen_hbm, out_hbm, row_v, buf_v, seq_v, vl_v):
    wid = lax.axis_index("s") * NC + lax.axis_index("c")
    pltpu.sync_copy(seqlen_hbm, seq_v.at[pl.ds(0, B)])
    lanes = lax.iota(jnp.int32, L)
    zeros_f = jnp.zeros((L,), jnp.float32)
    zeros_i = jnp.zeros((L,), jnp.int32)
    ones_i = jnp.ones((L,), jnp.int32)

    def row_body(i, vl_vec):
        row = wid * RPW + i
        pltpu.sync_copy(scores_hbm.at[row], row_v)
        s = seq_v[pl.ds(row, L)][0]
        s_vec = jnp.full((L,), s, jnp.int32)
        k = (s >> 4) + 1
        k_vec = jnp.full((L,), k, jnp.int32)
        nblk = (s + 127) >> 7

        def mask_body(jc, _):
            pos = lanes + jc * L
            d = row_v[pl.ds(jc * L, L)]
            row_v[pl.ds(jc * L, L)] = jnp.where(pos < s_vec, d, zeros_f)
            return 0
        lax.fori_loop(s >> 4, nblk * 8, mask_body, 0)

        def coarse_body(it, st):
            clo, chi = st
            mid = (clo + chi) >> 1
            t_vec = plsc.bitcast(jnp.full((L,), mid, jnp.int32), jnp.float32)

            def cnt_body(jb, acc):
                base = jb * (8 * L)
                for u in range(8):
                    d = row_v[pl.ds(base + u * L, L)]
                    acc = acc + jnp.where(d >= t_vec, ones_i, zeros_i)
                return acc
            cnt = jnp.sum(lax.fori_loop(0, nblk, cnt_body, zeros_i))
            ge = cnt >= k
            return jnp.where(ge, mid, clo), jnp.where(ge, chi, mid)
        lo, hi = lax.fori_loop(0, M1, coarse_body,
                               (jnp.int32(LO0), jnp.int32(HI0)))
        hi_vec = plsc.bitcast(jnp.full((L,), hi, jnp.int32), jnp.float32)
        lo_vec = plsc.bitcast(jnp.full((L,), lo, jnp.int32), jnp.float32)

        def band_body(jb, st):
            sacc, cacc, off = st
            base = jb * (8 * L)
            for u in range(8):
                d = row_v[pl.ds(base + u * L, L)]
                m_hi = d >= hi_vec
                m_band = (d >= lo_vec) & jnp.logical_not(m_hi)
                sacc = sacc + jnp.where(m_hi, d, zeros_f)
                cacc = cacc + jnp.where(m_hi, ones_i, zeros_i)
                buf_v[pl.ds(off, L)] = d
                pc = plsc.all_reduce_population_count(m_band)
                off = off + jnp.where(pc[0] > 0, L, 0)
            return sacc, cacc, off
        sacc, cacc, off = lax.fori_loop(
            0, nblk, band_body, (zeros_f, zeros_i, jnp.int32(0)))
        sum_hi = jnp.sum(sacc)
        cnt_hi = jnp.sum(cacc)
        nbc = off >> 4

        def search_body(it, st):
            clo, chi = st
            mid = (clo + chi) >> 1
            t_vec = plsc.bitcast(jnp.full((L,), mid, jnp.int32), jnp.float32)

            def cnt2_body(jb, acc):
                d = buf_v[pl.ds(jb * L, L)]
                m = (d >= t_vec) & (d < hi_vec)
                return acc + jnp.where(m, ones_i, zeros_i)
            cnt = cnt_hi + jnp.sum(lax.fori_loop(0, nbc, cnt2_body, zeros_i))
            ge = cnt >= k
            return jnp.where(ge, mid, clo), jnp.where(ge, chi, mid)
        lo2, _ = lax.fori_loop(0, TOTAL_ITERS - M1, search_body, (lo, hi))
        t_vec = plsc.bitcast(jnp.full((L,), lo2, jnp.int32), jnp.float32)

        def fin_body(jb, st):
            sa, ca = st
            d = buf_v[pl.ds(jb * L, L)]
            gt = (d > t_vec) & (d < hi_vec)
            return sa + jnp.where(gt, d, zeros_f), ca + jnp.where(gt, ones_i, zeros_i)
        sfin, cfin = lax.fori_loop(0, nbc, fin_body, (zeros_f, zeros_i))

        tot_vec = (jnp.full((L,), sum_hi) + jnp.full((L,), jnp.sum(sfin))
                   + (k_vec - jnp.full((L,), cnt_hi, jnp.int32)
                      - jnp.full((L,), jnp.sum(cfin), jnp.int32)
                      ).astype(jnp.float32) * t_vec)
        i_vec = jnp.full((L,), i, jnp.int32)
        vl_vec = jnp.where(lanes == i_vec, tot_vec, vl_vec)
        vl_vec = jnp.where(lanes == i_vec + RPW, k_vec.astype(jnp.float32), vl_vec)
        return vl_vec

    vl_v[...] = lax.fori_loop(0, RPW, row_body, zeros_f)
    pltpu.sync_copy(vl_v, out_hbm.at[wid])


_sc_topk = pl.kernel(
    _sc_body,
    out_type=jax.ShapeDtypeStruct((NW, L), jnp.float32),
    mesh=plsc.VectorSubcoreMesh(core_axis_name="c", subcore_axis_name="s"),
    scratch_types=[
        pltpu.VMEM((N,), jnp.float32),
        pltpu.VMEM((N,), jnp.float32),
        pltpu.VMEM((B + L,), jnp.int32),
        pltpu.VMEM((L,), jnp.float32),
    ],
    compiler_params=pltpu.CompilerParams(needs_layout_passes=False),
)


def _tc_bce_body(vl_ref, lab_ref, out_ref):
    raw = vl_ref[...]
    v = raw[:, :RPW] / raw[:, RPW:2 * RPW]
    lab = lab_ref[...]
    terms = lab * jnp.log(v) + (1.0 - lab) * jnp.log(1.0 - v)
    out_ref[0, 0] = -jnp.sum(terms) / B


_tc_bce = pl.pallas_call(
    _tc_bce_body,
    out_shape=jax.ShapeDtypeStruct((1, 1), jnp.float32),
    out_specs=pl.BlockSpec(memory_space=pltpu.SMEM),
)


@jax.jit
def kernel(scores, label, seqlen):
    vl_raw = _sc_topk(scores, seqlen)
    loss = _tc_bce(vl_raw, label.reshape(NW, RPW))
    return loss[0, 0]
